# Initial kernel scaffold; baseline (speedup 1.0000x reference)
import jax
import jax.numpy as jnp
from jax.experimental import pallas as pl


def kernel(tokens, W_E):
    raise NotImplementedError("write your pallas kernel here")


# speedup vs baseline: 1.4756x; 1.4756x over previous
import functools

import jax
import jax.numpy as jnp
from jax import lax
from jax.experimental import pallas as pl
from jax.experimental.pallas import tpu as pltpu
from jax.experimental.pallas import tpu_sc as plsc


@functools.cache
def _make_embed_gather(V, D, B):
    info = plsc.get_sparse_core_info()
    NC, NS = info.num_cores, info.num_subcores
    NW = NC * NS
    assert B % NW == 0
    b_per_w = B // NW
    CH = 64
    assert b_per_w % CH == 0 and CH <= 128
    NCH = b_per_w // CH

    mesh = plsc.VectorSubcoreMesh(core_axis_name="c", subcore_axis_name="s")

    @functools.partial(
        pl.kernel,
        mesh=mesh,
        out_type=jax.ShapeDtypeStruct((B, D), jnp.float32),
        scratch_types=[
            pltpu.VMEM((b_per_w,), jnp.int32),
            pltpu.VMEM((CH, D), jnp.float32),
            pltpu.VMEM((CH, D), jnp.float32),
            pltpu.SemaphoreType.DMA,
            pltpu.SemaphoreType.DMA,
            pltpu.SemaphoreType.DMA,
            pltpu.SemaphoreType.DMA,
        ],
    )
    def k(idx_hbm, table_hbm, out_hbm, idx_v, buf0, buf1, gs0, gs1, ws0, ws1):
        wid = lax.axis_index("s") * NC + lax.axis_index("c")
        base = wid * b_per_w
        pltpu.sync_copy(idx_hbm.at[pl.ds(base, b_per_w)], idx_v)

        bufs = (buf0, buf1)
        gsems = (gs0, gs1)
        wsems = (ws0, ws1)

        gathers = [None] * NCH
        writes = [None] * NCH
        gathers[0] = pltpu.async_copy(
            table_hbm.at[idx_v.at[pl.ds(0, CH)]], bufs[0], gsems[0]
        )
        for c in range(NCH):
            s = c % 2
            if c + 1 < NCH:
                if c - 1 >= 0:
                    writes[c - 1].wait()
                gathers[c + 1] = pltpu.async_copy(
                    table_hbm.at[idx_v.at[pl.ds((c + 1) * CH, CH)]],
                    bufs[1 - s],
                    gsems[1 - s],
                )
            gathers[c].wait()
            writes[c] = pltpu.async_copy(
                bufs[s], out_hbm.at[pl.ds(base + c * CH, CH)], wsems[s]
            )
        writes[NCH - 2].wait()
        writes[NCH - 1].wait()

    return k


def kernel(tokens, W_E):
    B, P = tokens.shape
    V, D = W_E.shape
    idx = tokens.reshape(-1).astype(jnp.int32)
    out = _make_embed_gather(V, D, B * P)(idx, W_E)
    return out.reshape(B, P, D)


# speedup vs baseline: 1.5046x; 1.0197x over previous
import functools

import jax
import jax.numpy as jnp
from jax import lax
from jax.experimental import pallas as pl
from jax.experimental.pallas import tpu as pltpu
from jax.experimental.pallas import tpu_sc as plsc


@functools.cache
def _make_embed_gather(V, D, B):
    info = plsc.get_sparse_core_info()
    NC, NS = info.num_cores, info.num_subcores
    NW = NC * NS
    assert B % NW == 0
    b_per_w = B // NW
    CH = 32
    NBUF = 4
    assert b_per_w % CH == 0 and CH <= 128
    NCH = b_per_w // CH

    mesh = plsc.VectorSubcoreMesh(core_axis_name="c", subcore_axis_name="s")

    @functools.partial(
        pl.kernel,
        mesh=mesh,
        out_type=jax.ShapeDtypeStruct((B, D), jnp.float32),
        scratch_types=[
            pltpu.VMEM((b_per_w,), jnp.int32),
        ]
        + [pltpu.VMEM((CH, D), jnp.float32)] * NBUF
        + [pltpu.SemaphoreType.DMA] * (2 * NBUF),
    )
    def k(idx_hbm, table_hbm, out_hbm, idx_v, *rest):
        bufs = rest[:NBUF]
        gsems = rest[NBUF : 2 * NBUF]
        wsems = rest[2 * NBUF :]

        wid = lax.axis_index("s") * NC + lax.axis_index("c")
        base = wid * b_per_w
        pltpu.sync_copy(idx_hbm.at[pl.ds(base, b_per_w)], idx_v)

        def gather(c):
            s = c % NBUF
            return pltpu.async_copy(
                table_hbm.at[idx_v.at[pl.ds(c * CH, CH)]], bufs[s], gsems[s]
            )

        K = NBUF - 1
        gathers = [None] * NCH
        writes = [None] * NCH
        for c in range(min(K, NCH)):
            gathers[c] = gather(c)
        for c in range(NCH):
            s = c % NBUF
            n = c + K
            if n < NCH:
                if n - NBUF >= 0:
                    writes[n - NBUF].wait()
                gathers[n] = gather(n)
            gathers[c].wait()
            writes[c] = pltpu.async_copy(
                bufs[s], out_hbm.at[pl.ds(base + c * CH, CH)], wsems[s]
            )
        for c in range(max(0, NCH - NBUF), NCH):
            writes[c].wait()

    return k


def kernel(tokens, W_E):
    B, P = tokens.shape
    V, D = W_E.shape
    idx = tokens.reshape(-1).astype(jnp.int32)
    out = _make_embed_gather(V, D, B * P)(idx, W_E)
    return out.reshape(B, P, D)
